# baseline (speedup 1.0000x reference)
import jax
import jax.numpy as jnp
import numpy as np
from jax import lax
from jax.experimental import pallas as pl
from jax.experimental.pallas import tpu as pltpu
from jax.experimental.pallas import tpu_sc as plsc

_A_IDX = [0, 0, 0, 0, 0, 0, 1, 1, 1, 1, 1, 2, 2, 2, 2, 1, 1, 1, 0, 0, 0]
_B_IDX = [0, 0, 0, 0, 0, 0, 1, 1, 1, 1, 1, 2, 2, 2, 2, 2, 2, 2, 2, 2, 1]
_C_IDX = [0, 1, 2, 1, 0, 0, 1, 2, 1, 0, 0, 2, 1, 0, 0, 1, 0, 0, 0, 0, 0]
_D_IDX = [0, 1, 2, 2, 2, 1, 1, 2, 2, 2, 1, 2, 2, 2, 1, 2, 2, 1, 2, 1, 1]


def _tables():
    flat = [27 * a + 9 * b + 3 * c + d
            for a, b, c, d in zip(_A_IDX, _B_IDX, _C_IDX, _D_IDX)]
    rows, cols = np.triu_indices(6)
    s2 = np.sqrt(2)
    m = np.array([[1, 1, 1, s2, s2, s2],
                  [1, 1, 1, s2, s2, s2],
                  [1, 1, 1, s2, s2, s2],
                  [s2, s2, s2, 2, 2, 2],
                  [s2, s2, s2, 2, 2, 2],
                  [s2, s2, s2, 2, 2, 2]], dtype=np.float32)
    comp_of = {}
    for k, (r, c) in enumerate(zip(rows, cols)):
        comp_of[(r, c)] = k
        comp_of[(c, r)] = k
    out_comp = [comp_of[(i, j)] for i in range(6) for j in range(6)]
    out_scale = [float(m[i, j]) for i in range(6) for j in range(6)]
    return flat, out_comp, out_scale

_FLAT, _OUT_COMP, _OUT_SCALE = _tables()

_NB = 500000
_S = 400
_IN_W = _S * 81
_OUT_W = _S * 36
_NCHUNK = _NB // _S
_NW = 32
_ITERS = -(-_NCHUNK // _NW)


def _body(c_hbm, out_hbm, in0, in1, ou0, ou1, si0, si1, so0, so1):
    wid = lax.axis_index("s") * 2 + lax.axis_index("c")
    lane = lax.iota(jnp.int32, 16)
    lane81 = lane * 81
    lane36 = lane * 36

    ins, outs = (in0, in1), (ou0, ou1)
    isems, osems = (si0, si1), (so0, so1)

    def in_dma(m, slot):
        base = (wid + m * _NW) * _IN_W
        return pltpu.async_copy(c_hbm.at[pl.ds(base, _IN_W)], ins[slot],
                                isems[slot])

    def out_dma(m, slot):
        base = (wid + m * _NW) * _OUT_W
        return pltpu.async_copy(outs[slot], out_hbm.at[pl.ds(base, _OUT_W)],
                                osems[slot])

    in_dma(0, 0)

    def iter_body(i, _):
        for b in range(2):
            m = 2 * i + b
            chunk = wid + m * _NW
            valid = chunk < _NCHUNK

            @pl.when(valid)
            def _():
                pltpu.make_async_copy(
                    c_hbm.at[pl.ds(chunk * _IN_W, _IN_W)], ins[b],
                    isems[b]).wait()

            @pl.when(wid + (m + 1) * _NW < _NCHUNK)
            def _():
                in_dma(m + 1, 1 - b)

            @pl.when(valid & (m >= 2))
            def _():
                base = (chunk - 2 * _NW) * _OUT_W
                pltpu.make_async_copy(
                    outs[b], out_hbm.at[pl.ds(base, _OUT_W)],
                    osems[b]).wait()

            @pl.when(valid)
            def _():
                def group_step(g, _):
                    s81 = lane81 + g * (16 * 81)
                    s36 = lane36 + g * (16 * 36)
                    vals = [plsc.load_gather(ins[b], [s81 + _FLAT[k]])
                            for k in range(21)]
                    for j in range(36):
                        plsc.store_scatter(
                            outs[b], [s36 + j],
                            vals[_OUT_COMP[j]] * _OUT_SCALE[j])
                    return 0

                lax.fori_loop(0, _S // 16, group_step, 0)
                out_dma(m, b)

        return 0

    lax.fori_loop(0, _ITERS // 2, iter_body, 0)

    for m in (_ITERS - 2, _ITERS - 1):
        chunk = wid + m * _NW

        @pl.when(chunk < _NCHUNK)
        def _():
            pltpu.make_async_copy(
                outs[m % 2], out_hbm.at[pl.ds(chunk * _OUT_W, _OUT_W)],
                osems[m % 2]).wait()


@jax.jit
def kernel(C):
    c2 = C.reshape(_NB * 81)
    mesh = plsc.VectorSubcoreMesh(core_axis_name="c", subcore_axis_name="s")
    out = pl.kernel(
        _body,
        out_type=jax.ShapeDtypeStruct((_NB * 36,), jnp.float32),
        mesh=mesh,
        scratch_types=[
            pltpu.VMEM((_IN_W,), jnp.float32),
            pltpu.VMEM((_IN_W,), jnp.float32),
            pltpu.VMEM((_OUT_W,), jnp.float32),
            pltpu.VMEM((_OUT_W,), jnp.float32),
            pltpu.SemaphoreType.DMA,
            pltpu.SemaphoreType.DMA,
            pltpu.SemaphoreType.DMA,
            pltpu.SemaphoreType.DMA,
        ],
        compiler_params=pltpu.CompilerParams(needs_layout_passes=False),
    )(c2)
    return out.reshape(_NB, 6, 6)


# speedup vs baseline: 1.0001x; 1.0001x over previous
import jax
import jax.numpy as jnp
import numpy as np
from jax import lax
from jax.experimental import pallas as pl
from jax.experimental.pallas import tpu as pltpu
from jax.experimental.pallas import tpu_sc as plsc

_A_IDX = [0, 0, 0, 0, 0, 0, 1, 1, 1, 1, 1, 2, 2, 2, 2, 1, 1, 1, 0, 0, 0]
_B_IDX = [0, 0, 0, 0, 0, 0, 1, 1, 1, 1, 1, 2, 2, 2, 2, 2, 2, 2, 2, 2, 1]
_C_IDX = [0, 1, 2, 1, 0, 0, 1, 2, 1, 0, 0, 2, 1, 0, 0, 1, 0, 0, 0, 0, 0]
_D_IDX = [0, 1, 2, 2, 2, 1, 1, 2, 2, 2, 1, 2, 2, 2, 1, 2, 2, 1, 2, 1, 1]


def _tables():
    flat = [27 * a + 9 * b + 3 * c + d
            for a, b, c, d in zip(_A_IDX, _B_IDX, _C_IDX, _D_IDX)]
    rows, cols = np.triu_indices(6)
    s2 = np.sqrt(2)
    m = np.array([[1, 1, 1, s2, s2, s2],
                  [1, 1, 1, s2, s2, s2],
                  [1, 1, 1, s2, s2, s2],
                  [s2, s2, s2, 2, 2, 2],
                  [s2, s2, s2, 2, 2, 2],
                  [s2, s2, s2, 2, 2, 2]], dtype=np.float32)
    comp_of = {}
    for k, (r, c) in enumerate(zip(rows, cols)):
        comp_of[(r, c)] = k
        comp_of[(c, r)] = k
    out_comp = [comp_of[(i, j)] for i in range(6) for j in range(6)]
    out_scale = [float(m[i, j]) for i in range(6) for j in range(6)]
    return flat, out_comp, out_scale

_FLAT, _OUT_COMP, _OUT_SCALE = _tables()

_NB = 500000
_S = 400
_IN_W = _S * 81
_OUT_W = _S * 36
_NCHUNK = _NB // _S
_NW = 32
_ITERS = -(-_NCHUNK // _NW)


def _body(c_hbm, out_hbm, in_v, out_v):
    wid = lax.axis_index("s") * 2 + lax.axis_index("c")
    lane = lax.iota(jnp.int32, 16)
    lane81 = lane * 81
    lane36 = lane * 36

    def chunk_step(i, _):
        chunk = wid + i * _NW

        @pl.when(chunk < _NCHUNK)
        def _():
            pltpu.sync_copy(c_hbm.at[pl.ds(chunk * _IN_W, _IN_W)], in_v)

            def group_step(g, _):
                s81 = lane81 + g * (16 * 81)
                s36 = lane36 + g * (16 * 36)
                vals = [plsc.load_gather(in_v, [s81 + _FLAT[k]])
                        for k in range(21)]
                for j in range(36):
                    plsc.store_scatter(
                        out_v, [s36 + j],
                        vals[_OUT_COMP[j]] * _OUT_SCALE[j])
                return 0

            lax.fori_loop(0, _S // 16, group_step, 0)
            pltpu.sync_copy(out_v, out_hbm.at[pl.ds(chunk * _OUT_W, _OUT_W)])

        return 0

    lax.fori_loop(0, _ITERS, chunk_step, 0)


@jax.jit
def kernel(C):
    c2 = C.reshape(_NB * 81)
    mesh = plsc.VectorSubcoreMesh(core_axis_name="c", subcore_axis_name="s")
    out = pl.kernel(
        _body,
        out_type=jax.ShapeDtypeStruct((_NB * 36,), jnp.float32),
        mesh=mesh,
        scratch_types=[
            pltpu.VMEM((_IN_W,), jnp.float32),
            pltpu.VMEM((_OUT_W,), jnp.float32),
        ],
        compiler_params=pltpu.CompilerParams(needs_layout_passes=False),
    )(c2)
    return out.reshape(_NB, 6, 6)


# speedup vs baseline: 42.3674x; 42.3645x over previous
import jax
import jax.numpy as jnp
import numpy as np
from jax import lax
from jax.experimental import pallas as pl
from jax.experimental.pallas import tpu as pltpu
from jax.experimental.pallas import tpu_sc as plsc

_A_IDX = [0, 0, 0, 0, 0, 0, 1, 1, 1, 1, 1, 2, 2, 2, 2, 1, 1, 1, 0, 0, 0]
_B_IDX = [0, 0, 0, 0, 0, 0, 1, 1, 1, 1, 1, 2, 2, 2, 2, 2, 2, 2, 2, 2, 1]
_C_IDX = [0, 1, 2, 1, 0, 0, 1, 2, 1, 0, 0, 2, 1, 0, 0, 1, 0, 0, 0, 0, 0]
_D_IDX = [0, 1, 2, 2, 2, 1, 1, 2, 2, 2, 1, 2, 2, 2, 1, 2, 2, 1, 2, 1, 1]


def _tables():
    flat = [27 * a + 9 * b + 3 * c + d
            for a, b, c, d in zip(_A_IDX, _B_IDX, _C_IDX, _D_IDX)]
    rows, cols = np.triu_indices(6)
    s2 = np.sqrt(2)
    m = np.array([[1, 1, 1, s2, s2, s2],
                  [1, 1, 1, s2, s2, s2],
                  [1, 1, 1, s2, s2, s2],
                  [s2, s2, s2, 2, 2, 2],
                  [s2, s2, s2, 2, 2, 2],
                  [s2, s2, s2, 2, 2, 2]], dtype=np.float32)
    comp_of = {}
    for k, (r, c) in enumerate(zip(rows, cols)):
        comp_of[(r, c)] = k
        comp_of[(c, r)] = k
    out_comp = [comp_of[(i, j)] for i in range(6) for j in range(6)]
    out_scale = [float(m[i, j]) for i in range(6) for j in range(6)]
    return flat, out_comp, out_scale

_FLAT, _OUT_COMP, _OUT_SCALE = _tables()

_NB = 500000
_S = 160
_NCHUNK = _NB // _S
_NW = 32
_ITERS = -(-_NCHUNK // _NW)


def _body(c_hbm, out_hbm, in0, in1, ou0, ou1, si0, si1, so0, so1):
    wid = lax.axis_index("s") * 2 + lax.axis_index("c")
    lane = lax.iota(jnp.int32, 16)

    ins, outs = (in0, in1), (ou0, ou1)
    isems, osems = (si0, si1), (so0, so1)

    def in_dma(m, slot):
        base = (wid + m * _NW) * _S
        return pltpu.async_copy(c_hbm.at[pl.ds(base, _S)], ins[slot],
                                isems[slot])

    def out_dma(m, slot):
        base = (wid + m * _NW) * _S
        return pltpu.async_copy(outs[slot], out_hbm.at[pl.ds(base, _S)],
                                osems[slot])

    in_dma(0, 0)

    def iter_body(i, _):
        for b in range(2):
            m = 2 * i + b
            chunk = wid + m * _NW
            valid = chunk < _NCHUNK

            @pl.when(valid)
            def _():
                pltpu.make_async_copy(
                    c_hbm.at[pl.ds(chunk * _S, _S)], ins[b], isems[b]).wait()

            @pl.when(wid + (m + 1) * _NW < _NCHUNK)
            def _():
                in_dma(m + 1, 1 - b)

            @pl.when(valid & (m >= 2))
            def _():
                base = (chunk - 2 * _NW) * _S
                pltpu.make_async_copy(
                    outs[b], out_hbm.at[pl.ds(base, _S)], osems[b]).wait()

            @pl.when(valid)
            def _():
                def group_step(g, _):
                    sidx = lane + g * 16
                    vals = [plsc.load_gather(
                                ins[b],
                                [sidx, jnp.full((16,), _FLAT[k], jnp.int32)])
                            for k in range(21)]
                    for j in range(36):
                        plsc.store_scatter(
                            outs[b], [sidx, jnp.full((16,), j, jnp.int32)],
                            vals[_OUT_COMP[j]] * _OUT_SCALE[j])
                    return 0

                lax.fori_loop(0, _S // 16, group_step, 0)
                out_dma(m, b)

        return 0

    lax.fori_loop(0, _ITERS // 2, iter_body, 0)

    for m in (_ITERS - 2, _ITERS - 1):
        chunk = wid + m * _NW

        @pl.when(chunk < _NCHUNK)
        def _():
            pltpu.make_async_copy(
                outs[m % 2], out_hbm.at[pl.ds(chunk * _S, _S)],
                osems[m % 2]).wait()


@jax.jit
def kernel(C):
    c2 = C.reshape(_NB, 81)
    mesh = plsc.VectorSubcoreMesh(core_axis_name="c", subcore_axis_name="s")
    out = pl.kernel(
        _body,
        out_type=jax.ShapeDtypeStruct((_NB, 36), jnp.float32),
        mesh=mesh,
        scratch_types=[
            pltpu.VMEM((_S, 81), jnp.float32),
            pltpu.VMEM((_S, 81), jnp.float32),
            pltpu.VMEM((_S, 36), jnp.float32),
            pltpu.VMEM((_S, 36), jnp.float32),
            pltpu.SemaphoreType.DMA,
            pltpu.SemaphoreType.DMA,
            pltpu.SemaphoreType.DMA,
            pltpu.SemaphoreType.DMA,
        ],
        compiler_params=pltpu.CompilerParams(needs_layout_passes=False),
    )(c2)
    return out.reshape(_NB, 6, 6)
